# baseline (speedup 1.0000x reference)
import functools

import jax
import jax.numpy as jnp
from jax import lax
from jax.experimental import pallas as pl
from jax.experimental.pallas import tpu as pltpu
from jax.experimental.pallas import tpu_sc as plsc

N = 10000
E = 320000
H = 128
C = 64

NC = 2
NS = 16
NW = NC * NS

K = 128
J = 80
EPAD = NW * J * K
NP_ = 10240
ROWS_PER_TILE = NP_ // NS
ZCHUNK = 128
NBUF = 4
GROUPS = J // NBUF

_mesh = plsc.VectorSubcoreMesh(
    core_axis_name="c", subcore_axis_name="s", num_cores=NC, num_subcores=NS)


def _zero_rows(rows_ref, nrows, width):
    zf = jnp.zeros((16,), jnp.float32)

    def body(i, _):
        for j in range(width // 16):
            rows_ref[i, pl.ds(j * 16, 16)] = zf
        return 0

    lax.fori_loop(0, nrows, body, 0)


@functools.partial(
    pl.kernel,
    out_type=jax.ShapeDtypeStruct((NC, NP_, 16), jnp.float32),
    mesh=_mesh,
    scratch_types=[
        pltpu.VMEM_SHARED((NP_, 16), jnp.float32),
        pltpu.VMEM((2, K), jnp.int32),
        pltpu.VMEM((2, K), jnp.int32),
        pltpu.VMEM((K, 16), jnp.float32),
        pltpu.SemaphoreType.DMA,
        pltpu.SemaphoreType.DMA,
    ],
)
def _sc_degree(ei4_hbm, out_hbm, acc_sh, idx0, idx1, ones_v, ssem0, ssem1):
    cid = lax.axis_index("c")
    sid = lax.axis_index("s")
    wid = sid * NC + cid

    _zero_rows(ones_v, K, 16)
    base_rows = sid * ROWS_PER_TILE
    for t in range(ROWS_PER_TILE // ZCHUNK):
        pltpu.sync_copy(ones_v.at[pl.ds(0, ZCHUNK)],
                        acc_sh.at[pl.ds(base_rows + t * ZCHUNK, ZCHUNK)])
    of = jnp.ones((16,), jnp.float32)

    def fill(i, _):
        ones_v[i, pl.ds(0, 16)] = of
        return 0

    lax.fori_loop(0, K, fill, 0)
    plsc.subcore_barrier()

    def pair(i, _):
        pltpu.sync_copy(ei4_hbm.at[wid, 2 * i], idx0)
        s0 = pltpu.async_copy(ones_v, acc_sh.at[idx0.at[1]], ssem0, add=True)
        pltpu.sync_copy(ei4_hbm.at[wid, 2 * i + 1], idx1)
        s1 = pltpu.async_copy(ones_v, acc_sh.at[idx1.at[1]], ssem1, add=True)
        s0.wait()
        s1.wait()
        return 0

    lax.fori_loop(0, J // 2, pair, 0)

    plsc.subcore_barrier()
    pltpu.sync_copy(acc_sh.at[pl.ds(base_rows, ROWS_PER_TILE)],
                    out_hbm.at[cid, pl.ds(base_rows, ROWS_PER_TILE)])


@functools.partial(
    pl.kernel,
    out_type=jax.ShapeDtypeStruct((NC, NP_, H), jnp.float32),
    mesh=_mesh,
    scratch_types=[
        pltpu.VMEM_SHARED((NP_, H), jnp.float32),
        pltpu.VMEM((2, K), jnp.int32),
        pltpu.VMEM((2, K), jnp.int32),
        pltpu.VMEM((K, H), jnp.float32),
        pltpu.VMEM((K, H), jnp.float32),
        pltpu.SemaphoreType.DMA,
        pltpu.SemaphoreType.DMA,
        pltpu.SemaphoreType.DMA,
        pltpu.SemaphoreType.DMA,
    ],
)
def _sc_aggregate(ei4_hbm, g_hbm, out_hbm, acc_sh, idx0, idx1, rows0, rows1,
                  gsem0, gsem1, ssem0, ssem1):
    cid = lax.axis_index("c")
    sid = lax.axis_index("s")
    wid = sid * NC + cid

    _zero_rows(rows0, ZCHUNK, H)
    base_rows = sid * ROWS_PER_TILE
    for t in range(ROWS_PER_TILE // ZCHUNK):
        pltpu.sync_copy(rows0.at[pl.ds(0, ZCHUNK)],
                        acc_sh.at[pl.ds(base_rows + t * ZCHUNK, ZCHUNK)])
    plsc.subcore_barrier()

    def pair(i, _):
        pltpu.sync_copy(ei4_hbm.at[wid, 2 * i], idx0)
        g0 = pltpu.async_copy(g_hbm.at[idx0.at[0]], rows0, gsem0)
        pltpu.sync_copy(ei4_hbm.at[wid, 2 * i + 1], idx1)
        g1 = pltpu.async_copy(g_hbm.at[idx1.at[0]], rows1, gsem1)
        g0.wait()
        s0 = pltpu.async_copy(rows0, acc_sh.at[idx0.at[1]], ssem0, add=True)
        g1.wait()
        s1 = pltpu.async_copy(rows1, acc_sh.at[idx1.at[1]], ssem1, add=True)
        s0.wait()
        s1.wait()
        return 0

    lax.fori_loop(0, J // 2, pair, 0)

    plsc.subcore_barrier()
    pltpu.sync_copy(acc_sh.at[pl.ds(base_rows, ROWS_PER_TILE)],
                    out_hbm.at[cid, pl.ds(base_rows, ROWS_PER_TILE)])


R = 1000


def _dinv_block(deg_ref):
    deg = deg_ref[0, :, 0:1] + deg_ref[1, :, 0:1] + 1.0
    return lax.rsqrt(deg)


def _tc_first_body(deg_ref, emb_ref, w_ref, o_ref):
    dinv = _dinv_block(deg_ref)
    h = jnp.dot(emb_ref[...], w_ref[...], preferred_element_type=jnp.float32)
    o_ref[...] = h * dinv


def _tc_mid_body(deg_ref, s_ref, g_ref, b_ref, w_ref, o_ref):
    dinv = _dinv_block(deg_ref)
    s = s_ref[0] + s_ref[1] + g_ref[...]
    h = jnp.maximum(s * dinv + b_ref[...], 0.0)
    o_ref[...] = jnp.dot(h, w_ref[...], preferred_element_type=jnp.float32) * dinv


def _tc_last_body(deg_ref, s_ref, g_ref, b_ref, w_ref, bo_ref, o_ref):
    dinv = _dinv_block(deg_ref)
    s = s_ref[0] + s_ref[1] + g_ref[...]
    h = jnp.maximum(s * dinv + b_ref[...], 0.0)
    o_ref[...] = (jnp.dot(h, w_ref[...], preferred_element_type=jnp.float32)
                  + bo_ref[...])


def _deg_spec():
    return pl.BlockSpec((NC, R, 16), lambda i: (0, i, 0))


def _row_spec(width):
    return pl.BlockSpec((R, width), lambda i: (i, 0))


def _part_spec():
    return pl.BlockSpec((NC, R, H), lambda i: (0, i, 0))


def _full_spec(shape):
    return pl.BlockSpec(shape, lambda i: tuple(0 for _ in shape))


def _tc_first(degp, emb, W1):
    return pl.pallas_call(
        _tc_first_body,
        grid=(N // R,),
        in_specs=[_deg_spec(), _row_spec(H), _full_spec((H, H))],
        out_specs=_row_spec(H),
        out_shape=jax.ShapeDtypeStruct((N, H), jnp.float32),
    )(degp, emb, W1)


def _tc_mid(degp, S, g, b, W):
    return pl.pallas_call(
        _tc_mid_body,
        grid=(N // R,),
        in_specs=[_deg_spec(), _part_spec(), _row_spec(H),
                  _full_spec((1, H)), _full_spec((H, H))],
        out_specs=_row_spec(H),
        out_shape=jax.ShapeDtypeStruct((N, H), jnp.float32),
    )(degp, S, g, b, W)


def _tc_last(degp, S, g, b, Wout, bout):
    return pl.pallas_call(
        _tc_last_body,
        grid=(N // R,),
        in_specs=[_deg_spec(), _part_spec(), _row_spec(H),
                  _full_spec((1, H)), _full_spec((H, C)), _full_spec((1, C))],
        out_specs=_row_spec(C),
        out_shape=jax.ShapeDtypeStruct((N, C), jnp.float32),
    )(degp, S, g, b, Wout, bout)


@jax.jit
def kernel(x, edge_index, emb, W1, b1, W2, b2, Wout, bout):
    del x
    pad = jnp.stack([jnp.zeros((EPAD - E,), jnp.int32),
                     jnp.full((EPAD - E,), N, jnp.int32)])
    ei4 = (jnp.concatenate([edge_index, pad], axis=1)
           .reshape(2, NW, J, K).transpose(1, 2, 0, 3))
    degp = _sc_degree(ei4)
    g1 = _tc_first(degp, emb, W1)
    S1 = _sc_aggregate(ei4, g1)
    g2 = _tc_mid(degp, S1, g1, b1.reshape(1, H), W2)
    S2 = _sc_aggregate(ei4, g2)
    return _tc_last(degp, S2, g2, b2.reshape(1, H), Wout, bout.reshape(1, C))


# speedup vs baseline: 2.5709x; 2.5709x over previous
import functools

import jax
import jax.numpy as jnp
from jax import lax
from jax.experimental import pallas as pl
from jax.experimental.pallas import tpu as pltpu
from jax.experimental.pallas import tpu_sc as plsc

N = 10000
E = 320000
H = 128
C = 64

NC = 2
NS = 16
NW = NC * NS

K = 128
J = 80
EPAD = NW * J * K
NP_ = 10240
ROWS_PER_TILE = NP_ // NS
ZCHUNK = 128
NBUF = 4
GROUPS = J // NBUF

_mesh = plsc.VectorSubcoreMesh(
    core_axis_name="c", subcore_axis_name="s", num_cores=NC, num_subcores=NS)


def _zero_rows(rows_ref, nrows, width):
    zf = jnp.zeros((16,), jnp.float32)

    def body(i, _):
        for j in range(width // 16):
            rows_ref[i, pl.ds(j * 16, 16)] = zf
        return 0

    lax.fori_loop(0, nrows, body, 0)


@functools.partial(
    pl.kernel,
    out_type=jax.ShapeDtypeStruct((NC, NP_, 16), jnp.float32),
    mesh=_mesh,
    scratch_types=[
        pltpu.VMEM_SHARED((NP_, 16), jnp.float32),
        pltpu.VMEM((2, K), jnp.int32),
        pltpu.VMEM((2, K), jnp.int32),
        pltpu.VMEM((K, 16), jnp.float32),
        pltpu.SemaphoreType.DMA,
        pltpu.SemaphoreType.DMA,
    ],
)
def _sc_degree(ei4_hbm, out_hbm, acc_sh, idx0, idx1, ones_v, ssem0, ssem1):
    cid = lax.axis_index("c")
    sid = lax.axis_index("s")
    wid = sid * NC + cid

    _zero_rows(ones_v, K, 16)
    base_rows = sid * ROWS_PER_TILE
    for t in range(ROWS_PER_TILE // ZCHUNK):
        pltpu.sync_copy(ones_v.at[pl.ds(0, ZCHUNK)],
                        acc_sh.at[pl.ds(base_rows + t * ZCHUNK, ZCHUNK)])
    of = jnp.ones((16,), jnp.float32)

    def fill(i, _):
        ones_v[i, pl.ds(0, 16)] = of
        return 0

    lax.fori_loop(0, K, fill, 0)
    plsc.subcore_barrier()

    def pair(i, _):
        pltpu.sync_copy(ei4_hbm.at[wid, 2 * i], idx0)
        s0 = pltpu.async_copy(ones_v, acc_sh.at[idx0.at[1]], ssem0, add=True)
        pltpu.sync_copy(ei4_hbm.at[wid, 2 * i + 1], idx1)
        s1 = pltpu.async_copy(ones_v, acc_sh.at[idx1.at[1]], ssem1, add=True)
        s0.wait()
        s1.wait()
        return 0

    lax.fori_loop(0, J // 2, pair, 0)

    plsc.subcore_barrier()
    pltpu.sync_copy(acc_sh.at[pl.ds(base_rows, ROWS_PER_TILE)],
                    out_hbm.at[cid, pl.ds(base_rows, ROWS_PER_TILE)])


@functools.partial(
    pl.kernel,
    out_type=jax.ShapeDtypeStruct((NC, NP_, H), jnp.float32),
    mesh=_mesh,
    scratch_types=[
        pltpu.VMEM_SHARED((NP_, H), jnp.float32),
        pltpu.VMEM((2, K), jnp.int32),
        pltpu.VMEM((2, K), jnp.int32),
        pltpu.VMEM((K, H), jnp.float32),
        pltpu.VMEM((K, H), jnp.float32),
        pltpu.SemaphoreType.DMA,
        pltpu.SemaphoreType.DMA,
        pltpu.SemaphoreType.DMA,
        pltpu.SemaphoreType.DMA,
    ],
)
def _sc_aggregate(ei4_hbm, g_hbm, out_hbm, acc_sh, idx0, idx1, rows0, rows1,
                  gsem0, gsem1, ssem0, ssem1):
    cid = lax.axis_index("c")
    sid = lax.axis_index("s")
    wid = sid * NC + cid

    _zero_rows(rows0, ZCHUNK, H)
    base_rows = sid * ROWS_PER_TILE
    for t in range(ROWS_PER_TILE // ZCHUNK):
        pltpu.sync_copy(rows0.at[pl.ds(0, ZCHUNK)],
                        acc_sh.at[pl.ds(base_rows + t * ZCHUNK, ZCHUNK)])
    plsc.subcore_barrier()

    def pair(i, _):
        pltpu.sync_copy(ei4_hbm.at[wid, 2 * i], idx0)
        g0 = pltpu.async_copy(g_hbm.at[idx0.at[0]], rows0, gsem0)
        pltpu.sync_copy(ei4_hbm.at[wid, 2 * i + 1], idx1)
        g1 = pltpu.async_copy(g_hbm.at[idx1.at[0]], rows1, gsem1)
        g0.wait()
        s0 = pltpu.async_copy(rows0, acc_sh.at[idx0.at[1]], ssem0, add=True)
        g1.wait()
        s1 = pltpu.async_copy(rows1, acc_sh.at[idx1.at[1]], ssem1, add=True)
        s0.wait()
        s1.wait()
        return 0

    lax.fori_loop(0, J // 2, pair, 0)

    plsc.subcore_barrier()
    pltpu.sync_copy(acc_sh.at[pl.ds(base_rows, ROWS_PER_TILE)],
                    out_hbm.at[cid, pl.ds(base_rows, ROWS_PER_TILE)])


R = 1000


def _dinv_block(deg_ref):
    deg = deg_ref[0, :, 0:1] + deg_ref[1, :, 0:1] + 1.0
    return lax.rsqrt(deg)


def _tc_first_body(deg_ref, emb_ref, w_ref, o_ref):
    dinv = _dinv_block(deg_ref)
    h = jnp.dot(emb_ref[...], w_ref[...], preferred_element_type=jnp.float32)
    o_ref[...] = h * dinv


def _tc_mid_body(deg_ref, s_ref, g_ref, b_ref, w_ref, o_ref):
    dinv = _dinv_block(deg_ref)
    s = s_ref[0] + s_ref[1] + g_ref[...]
    h = jnp.maximum(s * dinv + b_ref[...], 0.0)
    o_ref[...] = jnp.dot(h, w_ref[...], preferred_element_type=jnp.float32) * dinv


def _tc_last_body(deg_ref, s_ref, g_ref, b_ref, w_ref, bo_ref, o_ref):
    dinv = _dinv_block(deg_ref)
    s = s_ref[0] + s_ref[1] + g_ref[...]
    h = jnp.maximum(s * dinv + b_ref[...], 0.0)
    o_ref[...] = (jnp.dot(h, w_ref[...], preferred_element_type=jnp.float32)
                  + bo_ref[...])


def _deg_spec():
    return pl.BlockSpec((NC, R, 16), lambda i: (0, i, 0))


def _row_spec(width):
    return pl.BlockSpec((R, width), lambda i: (i, 0))


def _part_spec():
    return pl.BlockSpec((NC, R, H), lambda i: (0, i, 0))


def _full_spec(shape):
    return pl.BlockSpec(shape, lambda i: tuple(0 for _ in shape))


def _tc_first(degp, emb, W1):
    return pl.pallas_call(
        _tc_first_body,
        grid=(N // R,),
        in_specs=[_deg_spec(), _row_spec(H), _full_spec((H, H))],
        out_specs=_row_spec(H),
        out_shape=jax.ShapeDtypeStruct((N, H), jnp.float32),
    )(degp, emb, W1)


def _tc_mid(degp, S, g, b, W):
    return pl.pallas_call(
        _tc_mid_body,
        grid=(N // R,),
        in_specs=[_deg_spec(), _part_spec(), _row_spec(H),
                  _full_spec((1, H)), _full_spec((H, H))],
        out_specs=_row_spec(H),
        out_shape=jax.ShapeDtypeStruct((N, H), jnp.float32),
    )(degp, S, g, b, W)


def _tc_last(degp, S, g, b, Wout, bout):
    return pl.pallas_call(
        _tc_last_body,
        grid=(N // R,),
        in_specs=[_deg_spec(), _part_spec(), _row_spec(H),
                  _full_spec((1, H)), _full_spec((H, C)), _full_spec((1, C))],
        out_specs=_row_spec(C),
        out_shape=jax.ShapeDtypeStruct((N, C), jnp.float32),
    )(degp, S, g, b, Wout, bout)


@jax.jit
def kernel(x, edge_index, emb, W1, b1, W2, b2, Wout, bout):
    del x
    npad = EPAD - E
    r = jnp.arange(npad, dtype=jnp.int32)
    pad = jnp.stack([r % N, N + r % (NP_ - N)])
    ei4 = (jnp.concatenate([edge_index, pad], axis=1)
           .reshape(2, NW, J, K).transpose(1, 2, 0, 3))
    degp = _sc_degree(ei4)
    g1 = _tc_first(degp, emb, W1)
    S1 = _sc_aggregate(ei4, g1)
    g2 = _tc_mid(degp, S1, g1, b1.reshape(1, H), W2)
    S2 = _sc_aggregate(ei4, g2)
    return _tc_last(degp, S2, g2, b2.reshape(1, H), Wout, bout.reshape(1, C))
